# baseline (speedup 1.0000x reference)
import functools

import jax
import jax.numpy as jnp
from jax import lax
from jax.experimental import pallas as pl
from jax.experimental.pallas import tpu as pltpu
from jax.experimental.pallas import tpu_sc as plsc

N = 10000
E = 320000
D = 128
H = 128

NC = 2
NS = 16
NW = NC * NS

F32 = jnp.float32


_EPT = E // NS
_CH = 128
_NFULL = _EPT // _CH
_TAIL = _EPT - _NFULL * _CH


def _segsum_body(vals0, idx0, vals1, idx1, zeros, out0, out1,
                 idx_bufs, vals_bufs, idxt_v, valst_v, acc,
                 sl0, sl1):
    cid = lax.axis_index("c")
    sid = lax.axis_index("s")
    sls = (sl0, sl1)

    @pl.when(sid == 0)
    def _():
        pltpu.sync_copy(zeros, acc)

    plsc.subcore_barrier()

    def run(vals_hbm, idx_hbm):
        base0 = sid * _EPT

        def start_load(c, m):
            b = pl.multiple_of(base0 + c * _CH, 8)
            pltpu.async_copy(idx_hbm.at[pl.ds(b, _CH)], idx_bufs[m], sls[m])
            pltpu.async_copy(vals_hbm.at[pl.ds(b, _CH)], vals_bufs[m], sls[m])

        def wait_load(m):
            pltpu.make_async_copy(idx_hbm.at[pl.ds(0, _CH)],
                                  idx_bufs[m], sls[m]).wait()
            pltpu.make_async_copy(vals_hbm.at[pl.ds(0, _CH)],
                                  vals_bufs[m], sls[m]).wait()

        start_load(0, 0)

        def body(j, carry):
            for m in range(2):
                c = 2 * j + m
                wait_load(m)
                nc = c + 1

                @pl.when(nc <= _NFULL - 1)
                def _():
                    start_load(nc, (m + 1) % 2)

                pltpu.sync_copy(vals_bufs[m], acc.at[idx_bufs[m]], add=True)
            return carry

        lax.fori_loop(0, _NFULL // 2, body, 0)

        bt = pl.multiple_of(base0 + _NFULL * _CH, 8)
        pltpu.sync_copy(idx_hbm.at[pl.ds(bt, _TAIL)], idxt_v)
        pltpu.sync_copy(vals_hbm.at[pl.ds(bt, _TAIL)], valst_v)
        pltpu.sync_copy(valst_v, acc.at[idxt_v], add=True)

    @pl.when(cid == 0)
    def _():
        run(vals0, idx0)

    @pl.when(cid == 1)
    def _():
        run(vals1, idx1)

    plsc.subcore_barrier()

    @pl.when((sid == 0) & (cid == 0))
    def _():
        pltpu.sync_copy(acc, out0)

    @pl.when((sid == 0) & (cid == 1))
    def _():
        pltpu.sync_copy(acc, out1)


@functools.cache
def _segsum_fn():
    mesh = plsc.VectorSubcoreMesh(core_axis_name="c", subcore_axis_name="s")
    return pl.kernel(
        _segsum_body,
        out_type=[jax.ShapeDtypeStruct((N, D), F32),
                  jax.ShapeDtypeStruct((N, D), F32)],
        mesh=mesh,
        scratch_types=[
            [pltpu.VMEM((_CH,), jnp.int32) for _ in range(2)],
            [pltpu.VMEM((_CH, D), F32) for _ in range(2)],
            pltpu.VMEM((_TAIL,), jnp.int32),
            pltpu.VMEM((_TAIL, D), F32),
            pltpu.VMEM_SHARED((N, D), F32),
            pltpu.SemaphoreType.DMA,
            pltpu.SemaphoreType.DMA,
        ],
    )


def _segsum_pair(vals0, idx0, vals1, idx1, zeros):
    return _segsum_fn()(vals0, idx0, vals1, idx1, zeros)


_EPW = E // NW
_GNF = _EPW // _CH
_GTAIL = _EPW - _GNF * _CH


def _gather_body(table_s, table_t, src, tgt, gs_out, gt_out,
                 idx_bufs, rows_bufs, idxt_v, sem,
                 sl0, sl1, sl2, so0, so1, so2):
    cid = lax.axis_index("c")
    sid = lax.axis_index("s")
    wid = sid * NC + cid
    base0 = wid * _EPW
    sls = (sl0, sl1, sl2)
    sos = (so0, so1, so2)

    def run(table, idx_hbm, out_hbm):
        def start_idx(c, m):
            b = pl.multiple_of(base0 + c * _CH, 8)
            pltpu.async_copy(idx_hbm.at[pl.ds(b, _CH)], idx_bufs[m], sls[m])

        def wait_idx(m):
            pltpu.make_async_copy(idx_hbm.at[pl.ds(0, _CH)],
                                  idx_bufs[m], sls[m]).wait()

        def wait_out(m):
            pltpu.make_async_copy(rows_bufs[m],
                                  out_hbm.at[pl.ds(0, _CH)], sos[m]).wait()

        start_idx(0, 0)
        start_idx(1, 1)

        def body(j, carry):
            for m in range(3):
                c = 3 * j + m
                wait_idx(m)

                @pl.when(j >= 1)
                def _():
                    wait_out(m)

                pltpu.async_copy(table.at[idx_bufs[m]], rows_bufs[m],
                                 sem).wait()
                nc = c + 2

                @pl.when(nc <= _GNF - 1)
                def _():
                    start_idx(nc, (m + 2) % 3)

                b = pl.multiple_of(base0 + c * _CH, 8)
                pltpu.async_copy(rows_bufs[m], out_hbm.at[pl.ds(b, _CH)],
                                 sos[m])
            return carry

        lax.fori_loop(0, _GNF // 3, body, 0)
        for m in range(3):
            wait_out(m)

        bt = pl.multiple_of(base0 + _GNF * _CH, 8)
        pltpu.sync_copy(idx_hbm.at[pl.ds(bt, _GTAIL)], idxt_v)
        pltpu.async_copy(table.at[idxt_v], rows_bufs[0].at[pl.ds(0, _GTAIL)],
                         sem).wait()
        pltpu.sync_copy(rows_bufs[0].at[pl.ds(0, _GTAIL)],
                        out_hbm.at[pl.ds(bt, _GTAIL)])

    run(table_s, src, gs_out)
    run(table_t, tgt, gt_out)


def _gather_pair(table_s, table_t, src, tgt):
    mesh = plsc.VectorSubcoreMesh(core_axis_name="c", subcore_axis_name="s")
    w = table_s.shape[1]
    fn = pl.kernel(
        _gather_body,
        out_type=[jax.ShapeDtypeStruct((E, w), F32),
                  jax.ShapeDtypeStruct((E, w), F32)],
        mesh=mesh,
        scratch_types=[
            [pltpu.VMEM((_CH,), jnp.int32) for _ in range(3)],
            [pltpu.VMEM((_CH, w), F32) for _ in range(3)],
            pltpu.VMEM((_GTAIL,), jnp.int32),
            pltpu.SemaphoreType.DMA,
            pltpu.SemaphoreType.DMA,
            pltpu.SemaphoreType.DMA,
            pltpu.SemaphoreType.DMA,
            pltpu.SemaphoreType.DMA,
            pltpu.SemaphoreType.DMA,
            pltpu.SemaphoreType.DMA,
        ],
    )
    return fn(table_s, table_t, src, tgt)


_BE = 512
_BN = 1000


def _edge_alpha_body(e_ref, ws_ref, wt_ref, as_ref, at_ref):
    e = e_ref[...]
    as_ref[...] = jnp.exp(jnp.dot(e, ws_ref[...],
                                  preferred_element_type=F32))
    at_ref[...] = jnp.exp(jnp.dot(e, wt_ref[...],
                                  preferred_element_type=F32))


def _edge_alpha(edge_feat, Wsrc, Wtgt):
    grid = (E // _BE,)
    return pl.pallas_call(
        _edge_alpha_body,
        grid=grid,
        in_specs=[
            pl.BlockSpec((_BE, D), lambda i: (i, 0)),
            pl.BlockSpec((D, D), lambda i: (0, 0)),
            pl.BlockSpec((D, D), lambda i: (0, 0)),
        ],
        out_specs=[
            pl.BlockSpec((_BE, D), lambda i: (i, 0)),
            pl.BlockSpec((_BE, D), lambda i: (i, 0)),
        ],
        out_shape=[jax.ShapeDtypeStruct((E, D), F32),
                   jax.ShapeDtypeStruct((E, D), F32)],
    )(edge_feat, Wsrc, Wtgt)


def _edge_mlp_body(e_ref, gt_ref, gs_ref, was_ref, wat_ref,
                   wg_ref, wgt_ref, wgs_ref, wu_ref, wut_ref, wus_ref,
                   wo_ref, gam_ref, bet_ref, erw_ref,
                   sw_ref, tw_ref, eo_ref):
    bf = jnp.bfloat16
    e = e_ref[...]
    e16 = e.astype(bf)
    gt = gt_ref[...].astype(bf)
    gs = gs_ref[...].astype(bf)
    asrc = jnp.exp(jnp.dot(e16, was_ref[...].astype(bf),
                           preferred_element_type=F32))
    atgt = jnp.exp(jnp.dot(e16, wat_ref[...].astype(bf),
                           preferred_element_type=F32))
    g = (jnp.dot(e16, wg_ref[...].astype(bf), preferred_element_type=F32)
         + jnp.dot(gt, wgt_ref[...].astype(bf), preferred_element_type=F32)
         + jnp.dot(gs, wgs_ref[...].astype(bf), preferred_element_type=F32))
    u = (jnp.dot(e16, wu_ref[...].astype(bf), preferred_element_type=F32)
         + jnp.dot(gt, wut_ref[...].astype(bf), preferred_element_type=F32)
         + jnp.dot(gs, wus_ref[...].astype(bf), preferred_element_type=F32))
    h = g * jax.nn.sigmoid(g) * u
    y = jnp.dot(h.astype(bf), wo_ref[...].astype(bf),
                preferred_element_type=F32)
    mu = jnp.mean(y, axis=-1, keepdims=True)
    yc = y - mu
    var = jnp.mean(yc * yc, axis=-1, keepdims=True)
    ae = yc * jax.lax.rsqrt(var + 1e-5) * gam_ref[...] + bet_ref[...]
    sw_ref[...] = asrc * ae
    tw_ref[...] = atgt * ae
    eo_ref[...] = ae + erw_ref[...] * e


def _edge_mlp(edge_feat, Gt, Gs, Wsrc, Wtgt,
              Wg0, Wg_t, Wg_s, Wu0, Wu_t, Wu_s, Wo, gam, bet, erw):
    grid = (E // _BE,)
    return pl.pallas_call(
        _edge_mlp_body,
        grid=grid,
        in_specs=[
            pl.BlockSpec((_BE, D), lambda i: (i, 0)),
            pl.BlockSpec((_BE, D), lambda i: (i, 0)),
            pl.BlockSpec((_BE, D), lambda i: (i, 0)),
            pl.BlockSpec((D, D), lambda i: (0, 0)),
            pl.BlockSpec((D, D), lambda i: (0, 0)),
            pl.BlockSpec((D, H), lambda i: (0, 0)),
            pl.BlockSpec((D, H), lambda i: (0, 0)),
            pl.BlockSpec((D, H), lambda i: (0, 0)),
            pl.BlockSpec((D, H), lambda i: (0, 0)),
            pl.BlockSpec((D, H), lambda i: (0, 0)),
            pl.BlockSpec((D, H), lambda i: (0, 0)),
            pl.BlockSpec((H, D), lambda i: (0, 0)),
            pl.BlockSpec((1, D), lambda i: (0, 0)),
            pl.BlockSpec((1, D), lambda i: (0, 0)),
            pl.BlockSpec((1, D), lambda i: (0, 0)),
        ],
        out_specs=[
            pl.BlockSpec((_BE, D), lambda i: (i, 0)),
            pl.BlockSpec((_BE, D), lambda i: (i, 0)),
            pl.BlockSpec((_BE, D), lambda i: (i, 0)),
        ],
        out_shape=[jax.ShapeDtypeStruct((E, D), F32),
                   jax.ShapeDtypeStruct((E, D), F32),
                   jax.ShapeDtypeStruct((E, D), F32)],
    )(edge_feat, Gt, Gs, Wsrc, Wtgt,
      Wg0, Wg_t, Wg_s, Wu0, Wu_t, Wu_s, Wo, gam, bet, erw)


def _node_mlp_body(n_ref, asf_ref, atf_ref, ds_ref, dt_ref,
                   wg0_ref, wgt_ref, wgs_ref, wu0_ref, wut_ref, wus_ref,
                   wo_ref, gam_ref, bet_ref, nrw_ref, out_ref):
    n = n_ref[...]
    asf = asf_ref[...] / (ds_ref[...] + 1e-16)
    atf = atf_ref[...] / (dt_ref[...] + 1e-16)
    g = (jnp.dot(n, wg0_ref[...], preferred_element_type=F32)
         + jnp.dot(atf, wgt_ref[...], preferred_element_type=F32)
         + jnp.dot(asf, wgs_ref[...], preferred_element_type=F32))
    u = (jnp.dot(n, wu0_ref[...], preferred_element_type=F32)
         + jnp.dot(atf, wut_ref[...], preferred_element_type=F32)
         + jnp.dot(asf, wus_ref[...], preferred_element_type=F32))
    h = g * jax.nn.sigmoid(g) * u
    y = jnp.dot(h, wo_ref[...], preferred_element_type=F32)
    mu = jnp.mean(y, axis=-1, keepdims=True)
    yc = y - mu
    var = jnp.mean(yc * yc, axis=-1, keepdims=True)
    out = yc * jax.lax.rsqrt(var + 1e-5) * gam_ref[...] + bet_ref[...]
    out_ref[...] = out + nrw_ref[...] * n


def _node_mlp(node_feat, asf0, atf0, Dsrc, Dtgt,
              Wg0, Wg_t, Wg_s, Wu0, Wu_t, Wu_s, Wo, gam, bet, nrw):
    grid = (N // _BN,)
    blk = lambda i: (i, 0)
    zero = lambda i: (0, 0)
    return pl.pallas_call(
        _node_mlp_body,
        grid=grid,
        in_specs=[
            pl.BlockSpec((_BN, D), blk),
            pl.BlockSpec((_BN, D), blk),
            pl.BlockSpec((_BN, D), blk),
            pl.BlockSpec((_BN, D), blk),
            pl.BlockSpec((_BN, D), blk),
            pl.BlockSpec((D, H), zero),
            pl.BlockSpec((D, H), zero),
            pl.BlockSpec((D, H), zero),
            pl.BlockSpec((D, H), zero),
            pl.BlockSpec((D, H), zero),
            pl.BlockSpec((D, H), zero),
            pl.BlockSpec((H, D), zero),
            pl.BlockSpec((1, D), zero),
            pl.BlockSpec((1, D), zero),
            pl.BlockSpec((1, D), zero),
        ],
        out_specs=pl.BlockSpec((_BN, D), blk),
        out_shape=jax.ShapeDtypeStruct((N, D), F32),
    )(node_feat, asf0, atf0, Dsrc, Dtgt,
      Wg0, Wg_t, Wg_s, Wu0, Wu_t, Wu_s, Wo, gam, bet, nrw)


def kernel(node_feat, edge_feat, source_index, target_index,
           source_bincount, target_bincount, Wsrc, Wtgt, Weg, Weu, Weo,
           e_gamma, e_beta, Wng, Wnu, Wno, n_gamma, n_beta,
           node_res_weight, edge_res_weight):
    e_gamma2 = e_gamma.reshape(1, D)
    e_beta2 = e_beta.reshape(1, D)
    n_gamma2 = n_gamma.reshape(1, D)
    n_beta2 = n_beta.reshape(1, D)

    Weg0, Weg_t, Weg_s = Weg[0:D], Weg[D:2 * D], Weg[2 * D:3 * D]
    Weu0, Weu_t, Weu_s = Weu[0:D], Weu[D:2 * D], Weu[2 * D:3 * D]
    Wng0, Wng_t, Wng_s = Wng[0:D], Wng[D:2 * D], Wng[2 * D:3 * D]
    Wnu0, Wnu_t, Wnu_s = Wnu[0:D], Wnu[D:2 * D], Wnu[2 * D:3 * D]

    zeros = jnp.zeros((N, D), F32)

    Gs, Gt = _gather_pair(node_feat, node_feat, source_index, target_index)

    Asrc, Atgt = _edge_alpha(edge_feat, Wsrc, Wtgt)

    Dsrc, Dtgt = _segsum_pair(Asrc, source_index, Atgt, target_index, zeros)

    sw, tw, eo = _edge_mlp(edge_feat, Gt, Gs, Wsrc, Wtgt,
                           Weg0, Weg_t, Weg_s, Weu0, Weu_t, Weu_s,
                           Weo, e_gamma2, e_beta2, edge_res_weight)

    asf0, atf0 = _segsum_pair(sw, source_index, tw, target_index, zeros)

    attn_node = _node_mlp(node_feat, asf0, atf0, Dsrc, Dtgt,
                          Wng0, Wng_t, Wng_s, Wnu0, Wnu_t, Wnu_s,
                          Wno, n_gamma2, n_beta2, node_res_weight)

    return (attn_node, eo)


# speedup vs baseline: 1.4526x; 1.4526x over previous
import functools

import jax
import jax.numpy as jnp
from jax import lax
from jax.experimental import pallas as pl
from jax.experimental.pallas import tpu as pltpu
from jax.experimental.pallas import tpu_sc as plsc

N = 10000
E = 320000
D = 128
H = 128

NC = 2
NS = 16
NW = NC * NS

F32 = jnp.float32


_EPT = E // NS
_CH = 128
_NFULL = _EPT // _CH
_TAIL = _EPT - _NFULL * _CH


def _segsum_body(vals0, idx0, vals1, idx1, zeros, out0, out1,
                 idx_bufs, vals_bufs, idxt_v, valst_v, acc,
                 sl0, sl1):
    cid = lax.axis_index("c")
    sid = lax.axis_index("s")
    sls = (sl0, sl1)

    @pl.when(sid == 0)
    def _():
        pltpu.sync_copy(zeros, acc)

    plsc.subcore_barrier()

    def run(vals_hbm, idx_hbm):
        base0 = sid * _EPT

        def start_load(c, m):
            b = pl.multiple_of(base0 + c * _CH, 8)
            pltpu.async_copy(idx_hbm.at[pl.ds(b, _CH)], idx_bufs[m], sls[m])
            pltpu.async_copy(vals_hbm.at[pl.ds(b, _CH)], vals_bufs[m], sls[m])

        def wait_load(m):
            pltpu.make_async_copy(idx_hbm.at[pl.ds(0, _CH)],
                                  idx_bufs[m], sls[m]).wait()
            pltpu.make_async_copy(vals_hbm.at[pl.ds(0, _CH)],
                                  vals_bufs[m], sls[m]).wait()

        start_load(0, 0)

        def body(j, carry):
            for m in range(2):
                c = 2 * j + m
                wait_load(m)
                nc = c + 1

                @pl.when(nc <= _NFULL - 1)
                def _():
                    start_load(nc, (m + 1) % 2)

                pltpu.sync_copy(vals_bufs[m], acc.at[idx_bufs[m]], add=True)
            return carry

        lax.fori_loop(0, _NFULL // 2, body, 0)

        bt = pl.multiple_of(base0 + _NFULL * _CH, 8)
        pltpu.sync_copy(idx_hbm.at[pl.ds(bt, _TAIL)], idxt_v)
        pltpu.sync_copy(vals_hbm.at[pl.ds(bt, _TAIL)], valst_v)
        pltpu.sync_copy(valst_v, acc.at[idxt_v], add=True)

    @pl.when(cid == 0)
    def _():
        run(vals0, idx0)

    @pl.when(cid == 1)
    def _():
        run(vals1, idx1)

    plsc.subcore_barrier()

    @pl.when((sid == 0) & (cid == 0))
    def _():
        pltpu.sync_copy(acc, out0)

    @pl.when((sid == 0) & (cid == 1))
    def _():
        pltpu.sync_copy(acc, out1)


@functools.cache
def _segsum_fn():
    mesh = plsc.VectorSubcoreMesh(core_axis_name="c", subcore_axis_name="s")
    return pl.kernel(
        _segsum_body,
        out_type=[jax.ShapeDtypeStruct((N, D), F32),
                  jax.ShapeDtypeStruct((N, D), F32)],
        mesh=mesh,
        scratch_types=[
            [pltpu.VMEM((_CH,), jnp.int32) for _ in range(2)],
            [pltpu.VMEM((_CH, D), F32) for _ in range(2)],
            pltpu.VMEM((_TAIL,), jnp.int32),
            pltpu.VMEM((_TAIL, D), F32),
            pltpu.VMEM_SHARED((N, D), F32),
            pltpu.SemaphoreType.DMA,
            pltpu.SemaphoreType.DMA,
        ],
    )


def _segsum_pair(vals0, idx0, vals1, idx1, zeros):
    return _segsum_fn()(vals0, idx0, vals1, idx1, zeros)


_EPW = E // NW
_GNF = _EPW // _CH
_GTAIL = _EPW - _GNF * _CH


def _gather_body(table_s, table_t, src, tgt, gs_out, gt_out,
                 idx_bufs, rows_bufs, idxt_v, sem,
                 sl0, sl1, sl2, so0, so1, so2):
    cid = lax.axis_index("c")
    sid = lax.axis_index("s")
    wid = sid * NC + cid
    base0 = wid * _EPW
    sls = (sl0, sl1, sl2)
    sos = (so0, so1, so2)

    def run(table, idx_hbm, out_hbm):
        def start_idx(c, m):
            b = pl.multiple_of(base0 + c * _CH, 8)
            pltpu.async_copy(idx_hbm.at[pl.ds(b, _CH)], idx_bufs[m], sls[m])

        def wait_idx(m):
            pltpu.make_async_copy(idx_hbm.at[pl.ds(0, _CH)],
                                  idx_bufs[m], sls[m]).wait()

        def wait_out(m):
            pltpu.make_async_copy(rows_bufs[m],
                                  out_hbm.at[pl.ds(0, _CH)], sos[m]).wait()

        start_idx(0, 0)
        start_idx(1, 1)

        def body(j, carry):
            for m in range(3):
                c = 3 * j + m
                wait_idx(m)

                @pl.when(j >= 1)
                def _():
                    wait_out(m)

                pltpu.async_copy(table.at[idx_bufs[m]], rows_bufs[m],
                                 sem).wait()
                nc = c + 2

                @pl.when(nc <= _GNF - 1)
                def _():
                    start_idx(nc, (m + 2) % 3)

                b = pl.multiple_of(base0 + c * _CH, 8)
                pltpu.async_copy(rows_bufs[m], out_hbm.at[pl.ds(b, _CH)],
                                 sos[m])
            return carry

        lax.fori_loop(0, _GNF // 3, body, 0)
        for m in range(3):
            wait_out(m)

        bt = pl.multiple_of(base0 + _GNF * _CH, 8)
        pltpu.sync_copy(idx_hbm.at[pl.ds(bt, _GTAIL)], idxt_v)
        pltpu.async_copy(table.at[idxt_v], rows_bufs[0].at[pl.ds(0, _GTAIL)],
                         sem).wait()
        pltpu.sync_copy(rows_bufs[0].at[pl.ds(0, _GTAIL)],
                        out_hbm.at[pl.ds(bt, _GTAIL)])

    run(table_s, src, gs_out)
    run(table_t, tgt, gt_out)


def _gather_pair(table_s, table_t, src, tgt):
    mesh = plsc.VectorSubcoreMesh(core_axis_name="c", subcore_axis_name="s")
    w = table_s.shape[1]
    fn = pl.kernel(
        _gather_body,
        out_type=[jax.ShapeDtypeStruct((E, w), F32),
                  jax.ShapeDtypeStruct((E, w), F32)],
        mesh=mesh,
        scratch_types=[
            [pltpu.VMEM((_CH,), jnp.int32) for _ in range(3)],
            [pltpu.VMEM((_CH, w), F32) for _ in range(3)],
            pltpu.VMEM((_GTAIL,), jnp.int32),
            pltpu.SemaphoreType.DMA,
            pltpu.SemaphoreType.DMA,
            pltpu.SemaphoreType.DMA,
            pltpu.SemaphoreType.DMA,
            pltpu.SemaphoreType.DMA,
            pltpu.SemaphoreType.DMA,
            pltpu.SemaphoreType.DMA,
        ],
    )
    return fn(table_s, table_t, src, tgt)


_BE = 2048
_BN = 1000


def _edge_alpha_body(e_ref, ws_ref, wt_ref, as_ref, at_ref):
    e = e_ref[...]
    as_ref[...] = jnp.exp(jnp.dot(e, ws_ref[...],
                                  preferred_element_type=F32))
    at_ref[...] = jnp.exp(jnp.dot(e, wt_ref[...],
                                  preferred_element_type=F32))


def _edge_alpha(edge_feat, Wsrc, Wtgt):
    grid = (E // _BE,)
    return pl.pallas_call(
        _edge_alpha_body,
        grid=grid,
        in_specs=[
            pl.BlockSpec((_BE, D), lambda i: (i, 0)),
            pl.BlockSpec((D, D), lambda i: (0, 0)),
            pl.BlockSpec((D, D), lambda i: (0, 0)),
        ],
        out_specs=[
            pl.BlockSpec((_BE, D), lambda i: (i, 0)),
            pl.BlockSpec((_BE, D), lambda i: (i, 0)),
        ],
        out_shape=[jax.ShapeDtypeStruct((E, D), F32),
                   jax.ShapeDtypeStruct((E, D), F32)],
    )(edge_feat, Wsrc, Wtgt)


def _edge_mlp_body(e_ref, gt_ref, gs_ref, was_ref, wat_ref,
                   wg_ref, wgt_ref, wgs_ref, wu_ref, wut_ref, wus_ref,
                   wo_ref, gam_ref, bet_ref, erw_ref,
                   sw_ref, tw_ref, eo_ref):
    e = e_ref[...]
    gt = gt_ref[...]
    gs = gs_ref[...]
    asrc = jnp.exp(jnp.dot(e, was_ref[...], preferred_element_type=F32))
    atgt = jnp.exp(jnp.dot(e, wat_ref[...], preferred_element_type=F32))
    g = (jnp.dot(e, wg_ref[...], preferred_element_type=F32)
         + jnp.dot(gt, wgt_ref[...], preferred_element_type=F32)
         + jnp.dot(gs, wgs_ref[...], preferred_element_type=F32))
    u = (jnp.dot(e, wu_ref[...], preferred_element_type=F32)
         + jnp.dot(gt, wut_ref[...], preferred_element_type=F32)
         + jnp.dot(gs, wus_ref[...], preferred_element_type=F32))
    h = g * jax.nn.sigmoid(g) * u
    y = jnp.dot(h, wo_ref[...], preferred_element_type=F32)
    mu = jnp.mean(y, axis=-1, keepdims=True)
    yc = y - mu
    var = jnp.mean(yc * yc, axis=-1, keepdims=True)
    ae = yc * jax.lax.rsqrt(var + 1e-5) * gam_ref[...] + bet_ref[...]
    sw_ref[...] = asrc * ae
    tw_ref[...] = atgt * ae
    eo_ref[...] = ae + erw_ref[...] * e


def _edge_mlp(edge_feat, Gt, Gs, Wsrc, Wtgt,
              Wg0, Wg_t, Wg_s, Wu0, Wu_t, Wu_s, Wo, gam, bet, erw):
    grid = (E // _BE,)
    return pl.pallas_call(
        _edge_mlp_body,
        grid=grid,
        in_specs=[
            pl.BlockSpec((_BE, D), lambda i: (i, 0)),
            pl.BlockSpec((_BE, D), lambda i: (i, 0)),
            pl.BlockSpec((_BE, D), lambda i: (i, 0)),
            pl.BlockSpec((D, D), lambda i: (0, 0)),
            pl.BlockSpec((D, D), lambda i: (0, 0)),
            pl.BlockSpec((D, H), lambda i: (0, 0)),
            pl.BlockSpec((D, H), lambda i: (0, 0)),
            pl.BlockSpec((D, H), lambda i: (0, 0)),
            pl.BlockSpec((D, H), lambda i: (0, 0)),
            pl.BlockSpec((D, H), lambda i: (0, 0)),
            pl.BlockSpec((D, H), lambda i: (0, 0)),
            pl.BlockSpec((H, D), lambda i: (0, 0)),
            pl.BlockSpec((1, D), lambda i: (0, 0)),
            pl.BlockSpec((1, D), lambda i: (0, 0)),
            pl.BlockSpec((1, D), lambda i: (0, 0)),
        ],
        out_specs=[
            pl.BlockSpec((_BE, D), lambda i: (i, 0)),
            pl.BlockSpec((_BE, D), lambda i: (i, 0)),
            pl.BlockSpec((_BE, D), lambda i: (i, 0)),
        ],
        out_shape=[jax.ShapeDtypeStruct((E, D), F32),
                   jax.ShapeDtypeStruct((E, D), F32),
                   jax.ShapeDtypeStruct((E, D), F32)],
    )(edge_feat, Gt, Gs, Wsrc, Wtgt,
      Wg0, Wg_t, Wg_s, Wu0, Wu_t, Wu_s, Wo, gam, bet, erw)


def _node_mlp_body(n_ref, asf_ref, atf_ref, ds_ref, dt_ref,
                   wg0_ref, wgt_ref, wgs_ref, wu0_ref, wut_ref, wus_ref,
                   wo_ref, gam_ref, bet_ref, nrw_ref, out_ref):
    n = n_ref[...]
    asf = asf_ref[...] / (ds_ref[...] + 1e-16)
    atf = atf_ref[...] / (dt_ref[...] + 1e-16)
    g = (jnp.dot(n, wg0_ref[...], preferred_element_type=F32)
         + jnp.dot(atf, wgt_ref[...], preferred_element_type=F32)
         + jnp.dot(asf, wgs_ref[...], preferred_element_type=F32))
    u = (jnp.dot(n, wu0_ref[...], preferred_element_type=F32)
         + jnp.dot(atf, wut_ref[...], preferred_element_type=F32)
         + jnp.dot(asf, wus_ref[...], preferred_element_type=F32))
    h = g * jax.nn.sigmoid(g) * u
    y = jnp.dot(h, wo_ref[...], preferred_element_type=F32)
    mu = jnp.mean(y, axis=-1, keepdims=True)
    yc = y - mu
    var = jnp.mean(yc * yc, axis=-1, keepdims=True)
    out = yc * jax.lax.rsqrt(var + 1e-5) * gam_ref[...] + bet_ref[...]
    out_ref[...] = out + nrw_ref[...] * n


def _node_mlp(node_feat, asf0, atf0, Dsrc, Dtgt,
              Wg0, Wg_t, Wg_s, Wu0, Wu_t, Wu_s, Wo, gam, bet, nrw):
    grid = (N // _BN,)
    blk = lambda i: (i, 0)
    zero = lambda i: (0, 0)
    return pl.pallas_call(
        _node_mlp_body,
        grid=grid,
        in_specs=[
            pl.BlockSpec((_BN, D), blk),
            pl.BlockSpec((_BN, D), blk),
            pl.BlockSpec((_BN, D), blk),
            pl.BlockSpec((_BN, D), blk),
            pl.BlockSpec((_BN, D), blk),
            pl.BlockSpec((D, H), zero),
            pl.BlockSpec((D, H), zero),
            pl.BlockSpec((D, H), zero),
            pl.BlockSpec((D, H), zero),
            pl.BlockSpec((D, H), zero),
            pl.BlockSpec((D, H), zero),
            pl.BlockSpec((H, D), zero),
            pl.BlockSpec((1, D), zero),
            pl.BlockSpec((1, D), zero),
            pl.BlockSpec((1, D), zero),
        ],
        out_specs=pl.BlockSpec((_BN, D), blk),
        out_shape=jax.ShapeDtypeStruct((N, D), F32),
    )(node_feat, asf0, atf0, Dsrc, Dtgt,
      Wg0, Wg_t, Wg_s, Wu0, Wu_t, Wu_s, Wo, gam, bet, nrw)


def kernel(node_feat, edge_feat, source_index, target_index,
           source_bincount, target_bincount, Wsrc, Wtgt, Weg, Weu, Weo,
           e_gamma, e_beta, Wng, Wnu, Wno, n_gamma, n_beta,
           node_res_weight, edge_res_weight):
    e_gamma2 = e_gamma.reshape(1, D)
    e_beta2 = e_beta.reshape(1, D)
    n_gamma2 = n_gamma.reshape(1, D)
    n_beta2 = n_beta.reshape(1, D)

    Weg0, Weg_t, Weg_s = Weg[0:D], Weg[D:2 * D], Weg[2 * D:3 * D]
    Weu0, Weu_t, Weu_s = Weu[0:D], Weu[D:2 * D], Weu[2 * D:3 * D]
    Wng0, Wng_t, Wng_s = Wng[0:D], Wng[D:2 * D], Wng[2 * D:3 * D]
    Wnu0, Wnu_t, Wnu_s = Wnu[0:D], Wnu[D:2 * D], Wnu[2 * D:3 * D]

    zeros = jnp.zeros((N, D), F32)

    Gs, Gt = _gather_pair(node_feat, node_feat, source_index, target_index)

    Asrc, Atgt = _edge_alpha(edge_feat, Wsrc, Wtgt)

    Dsrc, Dtgt = _segsum_pair(Asrc, source_index, Atgt, target_index, zeros)

    sw, tw, eo = _edge_mlp(edge_feat, Gt, Gs, Wsrc, Wtgt,
                           Weg0, Weg_t, Weg_s, Weu0, Weu_t, Weu_s,
                           Weo, e_gamma2, e_beta2, edge_res_weight)

    asf0, atf0 = _segsum_pair(sw, source_index, tw, target_index, zeros)

    attn_node = _node_mlp(node_feat, asf0, atf0, Dsrc, Dtgt,
                          Wng0, Wng_t, Wng_s, Wnu0, Wnu_t, Wnu_s,
                          Wno, n_gamma2, n_beta2, node_res_weight)

    return (attn_node, eo)
